# baseline (speedup 1.0000x reference)
import functools

import jax
import jax.numpy as jnp
from jax import lax
from jax.experimental import pallas as pl
from jax.experimental.pallas import tpu as pltpu
from jax.experimental.pallas import tpu_sc as plsc

N = 10000
NP = 10240
D = 128
E = 320000
EP2 = 320000
NC = 2
NS = 16
NW = NC * NS
CH = 128
NCH = 80
EPT = CH * NCH
EPAD = NW * EPT
RPT = NP // NS
ZR = 128

_mesh = plsc.VectorSubcoreMesh(core_axis_name="c", subcore_axis_name="s")
_sc_params = pltpu.CompilerParams(use_tc_tiling_on_sc=False)


def _sc_deg(dst3):
    @functools.partial(
        pl.kernel,
        out_type=jax.ShapeDtypeStruct((NC, NP, 16), jnp.float32),
        mesh=_mesh,
        scratch_types=[
            pltpu.VMEM((ZR, 16), jnp.float32),
            pltpu.VMEM((CH, 16), jnp.float32),
            pltpu.VMEM((NCH, CH), jnp.int32),
            pltpu.VMEM_SHARED((NP, 16), jnp.float32),
            pltpu.SemaphoreType.DMA,
        ],
        compiler_params=_sc_params,
    )
    def k(dst_hbm, out_hbm, zbuf, ones_v, didx, acc, sem):
        c = lax.axis_index("c")
        s = lax.axis_index("s")
        wid = c * NS + s
        zero16 = jnp.zeros((16,), jnp.float32)
        one16 = jnp.ones((16,), jnp.float32)

        @pl.loop(0, ZR)
        def _(i):
            zbuf[i, :] = zero16

        @pl.loop(0, CH)
        def _(i):
            ones_v[i, :] = one16

        pltpu.sync_copy(dst_hbm.at[wid], didx)
        for kk in range(RPT // ZR):
            pltpu.sync_copy(zbuf, acc.at[pl.ds(s * RPT + kk * ZR, ZR)])
        plsc.subcore_barrier()

        @pl.loop(0, NCH // 8)
        def _(t):
            for b in range(8):
                pltpu.async_copy(ones_v, acc.at[didx.at[t * 8 + b]], sem, add=True)
            for b in range(8):
                pltpu.make_async_copy(ones_v, acc.at[didx.at[t * 8 + b]], sem).wait()

        plsc.subcore_barrier()
        for kk in range(RPT // ZR):
            r0 = s * RPT + kk * ZR
            pltpu.sync_copy(acc.at[pl.ds(r0, ZR)], out_hbm.at[c].at[pl.ds(r0, ZR)])

    return k(dst3)


def _sc_conv(h, src3, dst3):
    @functools.partial(
        pl.kernel,
        out_type=jax.ShapeDtypeStruct((NC, NP, D), jnp.float32),
        mesh=_mesh,
        scratch_types=[
            pltpu.VMEM((64, D), jnp.float32),
            pltpu.VMEM((CH, D), jnp.float32),
            pltpu.VMEM((CH, D), jnp.float32),
            pltpu.VMEM((CH,), jnp.int32),
            pltpu.VMEM((CH,), jnp.int32),
            pltpu.VMEM((CH,), jnp.int32),
            pltpu.VMEM((CH,), jnp.int32),
            pltpu.VMEM_SHARED((NP, D), jnp.float32),
            pltpu.SemaphoreType.DMA,
            pltpu.SemaphoreType.DMA,
            pltpu.SemaphoreType.DMA,
            pltpu.SemaphoreType.DMA,
        ],
        compiler_params=_sc_params,
    )
    def k(h_hbm, src_hbm, dst_hbm, out_hbm, zbuf, rows0, rows1,
          is0, is1, id0, id1, acc, sg0, sg1, si0, si1):
        c = lax.axis_index("c")
        s = lax.axis_index("s")
        wid = c * NS + s
        zero16 = jnp.zeros((16,), jnp.float32)
        rows = (rows0, rows1)
        isb = (is0, is1)
        idb = (id0, id1)
        sg = (sg0, sg1)
        si = (si0, si1)

        @pl.loop(0, 64)
        def _(i):
            @pl.loop(0, D, step=16)
            def _(q):
                zbuf[i, pl.ds(q, 16)] = zero16

        for kk in range(RPT // 64):
            pltpu.sync_copy(zbuf, acc.at[pl.ds(s * RPT + kk * 64, 64)])
        plsc.subcore_barrier()

        def issue_idx(j, p):
            pltpu.async_copy(src_hbm.at[wid].at[j], isb[p], si[p])
            pltpu.async_copy(dst_hbm.at[wid].at[j], idb[p], si[p])

        def wait_idx(p):
            pltpu.make_async_copy(src_hbm.at[wid].at[0], isb[p], si[p]).wait()
            pltpu.make_async_copy(dst_hbm.at[wid].at[0], idb[p], si[p]).wait()

        def issue_gather(p):
            pltpu.async_copy(h_hbm.at[isb[p]], rows[p], sg[p])

        def wait_gather(p):
            pltpu.make_async_copy(h_hbm.at[isb[p]], rows[p], sg[p]).wait()

        issue_idx(0, 0)
        wait_idx(0)
        issue_gather(0)
        issue_idx(1, 1)

        def step(j, p, next_gather, idx_prefetch):
            q = 1 - p
            wait_gather(p)
            if next_gather:
                wait_idx(q)
                issue_gather(q)
            pltpu.sync_copy(rows[p], acc.at[idb[p]], add=True)
            if idx_prefetch:
                issue_idx(j + 2, p)

        @pl.loop(0, NCH // 2 - 1)
        def _(t):
            step(2 * t, 0, True, True)
            step(2 * t + 1, 1, True, True)

        step(NCH - 2, 0, True, False)
        step(NCH - 1, 1, False, False)

        plsc.subcore_barrier()
        for kk in range(RPT // 64):
            r0 = s * RPT + kk * 64
            pltpu.sync_copy(acc.at[pl.ds(r0, 64)], out_hbm.at[c].at[pl.ds(r0, 64)])

    return k(h, src3, dst3)


def _sc_decode(ta, tb, i03, i13):
    @functools.partial(
        pl.kernel,
        out_type=jax.ShapeDtypeStruct((EPAD // 8, 128), jnp.float32),
        mesh=_mesh,
        scratch_types=[
            pltpu.VMEM((CH, 16), jnp.float32),
            pltpu.VMEM((CH, 16), jnp.float32),
            pltpu.VMEM((CH, 16), jnp.float32),
            pltpu.VMEM((CH, 16), jnp.float32),
            pltpu.VMEM((CH // 8, 128), jnp.float32),
            pltpu.VMEM((CH // 8, 128), jnp.float32),
            pltpu.VMEM((NCH, CH), jnp.int32),
            pltpu.VMEM((NCH, CH), jnp.int32),
            pltpu.SemaphoreType.DMA,
            pltpu.SemaphoreType.DMA,
            pltpu.SemaphoreType.DMA,
            pltpu.SemaphoreType.DMA,
            pltpu.SemaphoreType.DMA,
            pltpu.SemaphoreType.DMA,
        ],
        compiler_params=_sc_params,
    )
    def k(ta_hbm, tb_hbm, i0_hbm, i1_hbm, out_hbm, va0, vb0, va1, vb1,
          ob0, ob1, idxa, idxb, sa0, sb0, sa1, sb1, sw0, sw1):
        c = lax.axis_index("c")
        s = lax.axis_index("s")
        wid = c * NS + s
        base = wid * EPT

        pltpu.sync_copy(i0_hbm.at[wid], idxa)
        pltpu.sync_copy(i1_hbm.at[wid], idxb)

        va = (va0, va1)
        vb = (vb0, vb1)
        ob = (ob0, ob1)
        sa = (sa0, sa1)
        sb = (sb0, sb1)
        sw = (sw0, sw1)

        def out_slice(j):
            return out_hbm.at[pl.ds((base + j * CH) // 8, CH // 8)]

        def step(j, p, issue_next, guard):
            q = 1 - p
            pltpu.make_async_copy(ta_hbm.at[idxa.at[j]], va[p], sa[p]).wait()
            pltpu.make_async_copy(tb_hbm.at[idxb.at[j]], vb[p], sb[p]).wait()
            if issue_next:
                pltpu.async_copy(ta_hbm.at[idxa.at[j + 1]], va[q], sa[q])
                pltpu.async_copy(tb_hbm.at[idxb.at[j + 1]], vb[q], sb[q])

            def _wait_prev_write():
                pltpu.make_async_copy(ob[p], out_slice(j), sw[p]).wait()

            if guard is None:
                _wait_prev_write()
            else:
                pl.when(guard)(_wait_prev_write)

            @pl.loop(0, CH // 8)
            def _(r):
                for u in range(8):
                    ob[p][r, pl.ds(16 * u, 16)] = va[p][8 * r + u, :] + vb[p][8 * r + u, :]

            pltpu.async_copy(ob[p], out_slice(j), sw[p])

        pltpu.async_copy(ta_hbm.at[idxa.at[0]], va0, sa0)
        pltpu.async_copy(tb_hbm.at[idxb.at[0]], vb0, sb0)

        @pl.loop(0, NCH // 2 - 1)
        def _(t):
            step(2 * t, 0, True, t > 0)
            step(2 * t + 1, 1, True, t > 0)

        step(NCH - 2, 0, True, None)
        step(NCH - 1, 1, False, None)
        pltpu.make_async_copy(ob0, out_slice(NCH - 2), sw0).wait()
        pltpu.make_async_copy(ob1, out_slice(NCH - 1), sw1).wait()

    return k(ta, tb, i03, i13)


_R = 1024


def _tc_prep(degpair, xp, W1):
    def body(dp_ref, x_ref, w_ref, h_ref, rb_ref):
        deg = dp_ref[0][:, 0:1] + dp_ref[1][:, 0:1]
        r = jnp.where(deg > 0, lax.rsqrt(jnp.maximum(deg, 1.0)), 0.0)
        rb = jnp.broadcast_to(r, (_R, D))
        h = jnp.dot(x_ref[...], w_ref[...], preferred_element_type=jnp.float32)
        h_ref[...] = h * rb
        rb_ref[...] = rb

    return pl.pallas_call(
        body,
        grid=(NP // _R,),
        in_specs=[
            pl.BlockSpec((NC, _R, 16), lambda i: (0, i, 0)),
            pl.BlockSpec((_R, D), lambda i: (i, 0)),
            pl.BlockSpec((D, D), lambda i: (0, 0)),
        ],
        out_specs=[
            pl.BlockSpec((_R, D), lambda i: (i, 0)),
            pl.BlockSpec((_R, D), lambda i: (i, 0)),
        ],
        out_shape=[
            jax.ShapeDtypeStruct((NP, D), jnp.float32),
            jax.ShapeDtypeStruct((NP, D), jnp.float32),
        ],
    )(degpair, xp, W1)


def _tc_mid(zpair, rb, W2):
    def body(zp_ref, rb_ref, w_ref, h_ref):
        rbv = rb_ref[...]
        z = jnp.maximum((zp_ref[0] + zp_ref[1]) * rbv, 0.0)
        h_ref[...] = jnp.dot(z, w_ref[...], preferred_element_type=jnp.float32) * rbv

    return pl.pallas_call(
        body,
        grid=(NP // _R,),
        in_specs=[
            pl.BlockSpec((NC, _R, D), lambda i: (0, i, 0)),
            pl.BlockSpec((_R, D), lambda i: (i, 0)),
            pl.BlockSpec((D, D), lambda i: (0, 0)),
        ],
        out_specs=pl.BlockSpec((_R, D), lambda i: (i, 0)),
        out_shape=jax.ShapeDtypeStruct((NP, D), jnp.float32),
    )(zpair, rb, W2)


def _tc_final(zpair, rb, WA, WB):
    def body(zp_ref, rb_ref, wa_ref, wb_ref, ta_ref, tb_ref):
        z = (zp_ref[0] + zp_ref[1]) * rb_ref[...]
        ta_ref[...] = jnp.dot(z, wa_ref[...], preferred_element_type=jnp.float32)
        tb_ref[...] = jnp.dot(z, wb_ref[...], preferred_element_type=jnp.float32)

    return pl.pallas_call(
        body,
        grid=(NP // _R,),
        in_specs=[
            pl.BlockSpec((NC, _R, D), lambda i: (0, i, 0)),
            pl.BlockSpec((_R, D), lambda i: (i, 0)),
            pl.BlockSpec((D, 16), lambda i: (0, 0)),
            pl.BlockSpec((D, 16), lambda i: (0, 0)),
        ],
        out_specs=[
            pl.BlockSpec((_R, 16), lambda i: (i, 0)),
            pl.BlockSpec((_R, 16), lambda i: (i, 0)),
        ],
        out_shape=[
            jax.ShapeDtypeStruct((NP, 16), jnp.float32),
            jax.ShapeDtypeStruct((NP, 16), jnp.float32),
        ],
    )(zpair, rb, WA, WB)


def _pad_edges(idx, fill):
    return jnp.concatenate([idx, fill]).reshape(NW, NCH, CH)


def kernel(x, edge_index, pos_edge_index, neg_edge_index, W1, W2, Wlin):
    pad_nodes = N + (jnp.arange(EPAD - E, dtype=jnp.int32) % (NP - N))
    src3 = _pad_edges(edge_index[0], pad_nodes)
    dst3 = _pad_edges(edge_index[1], pad_nodes)
    pad_rows = jnp.arange(EPAD - EP2, dtype=jnp.int32) % N
    i03 = jnp.concatenate(
        [pos_edge_index[0], neg_edge_index[0], pad_rows]).reshape(NW, NCH, CH)
    i13 = jnp.concatenate(
        [pos_edge_index[1], neg_edge_index[1], pad_rows]).reshape(NW, NCH, CH)
    xp = jnp.pad(x, ((0, NP - N), (0, 0)))
    WA = jnp.pad(Wlin[:, :D].T, ((0, 0), (0, 14)))
    WB = jnp.pad(Wlin[:, D:].T, ((0, 0), (0, 14)))

    degpair = _sc_deg(dst3)
    h1, rb = _tc_prep(degpair, xp, W1)
    z1p = _sc_conv(h1, src3, dst3)
    h2 = _tc_mid(z1p, rb, W2)
    z2p = _sc_conv(h2, src3, dst3)
    ta, tb = _tc_final(z2p, rb, WA, WB)
    outp = _sc_decode(ta, tb, i03, i13)
    return outp[: EP2 * 2 // 16].reshape(EP2 // 8, 8, 16)[:, :, :2].reshape(EP2, 2)


# speedup vs baseline: 1.0257x; 1.0257x over previous
import functools

import jax
import jax.numpy as jnp
from jax import lax
from jax.experimental import pallas as pl
from jax.experimental.pallas import tpu as pltpu
from jax.experimental.pallas import tpu_sc as plsc

N = 10000
NP = 10240
D = 128
E = 320000
EP2 = 320000
NC = 2
NS = 16
NW = NC * NS
CH = 128
NCH = 80
EPT = CH * NCH
EPAD = NW * EPT
RPT = NP // NS
ZR = 128

_mesh = plsc.VectorSubcoreMesh(core_axis_name="c", subcore_axis_name="s")
_sc_params = pltpu.CompilerParams(use_tc_tiling_on_sc=False)


def _sc_deg(dst3):
    @functools.partial(
        pl.kernel,
        out_type=jax.ShapeDtypeStruct((NC, NP, 16), jnp.float32),
        mesh=_mesh,
        scratch_types=[
            pltpu.VMEM((ZR, 16), jnp.float32),
            pltpu.VMEM((CH, 16), jnp.float32),
            pltpu.VMEM((NCH, CH), jnp.int32),
            pltpu.VMEM_SHARED((NP, 16), jnp.float32),
            pltpu.SemaphoreType.DMA,
        ],
        compiler_params=_sc_params,
    )
    def k(dst_hbm, out_hbm, zbuf, ones_v, didx, acc, sem):
        c = lax.axis_index("c")
        s = lax.axis_index("s")
        wid = c * NS + s
        zero16 = jnp.zeros((16,), jnp.float32)
        one16 = jnp.ones((16,), jnp.float32)

        @pl.loop(0, ZR)
        def _(i):
            zbuf[i, :] = zero16

        @pl.loop(0, CH)
        def _(i):
            ones_v[i, :] = one16

        pltpu.sync_copy(dst_hbm.at[wid], didx)
        for kk in range(RPT // ZR):
            pltpu.sync_copy(zbuf, acc.at[pl.ds(s * RPT + kk * ZR, ZR)])
        plsc.subcore_barrier()

        @pl.loop(0, NCH // 8)
        def _(t):
            for b in range(8):
                pltpu.async_copy(ones_v, acc.at[didx.at[t * 8 + b]], sem, add=True)
            for b in range(8):
                pltpu.make_async_copy(ones_v, acc.at[didx.at[t * 8 + b]], sem).wait()

        plsc.subcore_barrier()
        for kk in range(RPT // ZR):
            r0 = s * RPT + kk * ZR
            pltpu.sync_copy(acc.at[pl.ds(r0, ZR)], out_hbm.at[c].at[pl.ds(r0, ZR)])

    return k(dst3)


def _sc_conv(h, src3, dst3):
    @functools.partial(
        pl.kernel,
        out_type=jax.ShapeDtypeStruct((NC, NP, D), jnp.float32),
        mesh=_mesh,
        scratch_types=[
            pltpu.VMEM((64, D), jnp.float32),
            pltpu.VMEM((CH, D), jnp.float32),
            pltpu.VMEM((CH, D), jnp.float32),
            pltpu.VMEM((CH,), jnp.int32),
            pltpu.VMEM((CH,), jnp.int32),
            pltpu.VMEM((CH,), jnp.int32),
            pltpu.VMEM((CH,), jnp.int32),
            pltpu.VMEM_SHARED((NP, D), jnp.float32),
            pltpu.SemaphoreType.DMA,
            pltpu.SemaphoreType.DMA,
            pltpu.SemaphoreType.DMA,
            pltpu.SemaphoreType.DMA,
        ],
        compiler_params=_sc_params,
    )
    def k(h_hbm, src_hbm, dst_hbm, out_hbm, zbuf, rows0, rows1,
          is0, is1, id0, id1, acc, sg0, sg1, si0, si1):
        c = lax.axis_index("c")
        s = lax.axis_index("s")
        wid = c * NS + s
        zero16 = jnp.zeros((16,), jnp.float32)
        rows = (rows0, rows1)
        isb = (is0, is1)
        idb = (id0, id1)
        sg = (sg0, sg1)
        si = (si0, si1)

        @pl.loop(0, 64)
        def _(i):
            @pl.loop(0, D, step=16)
            def _(q):
                zbuf[i, pl.ds(q, 16)] = zero16

        for kk in range(RPT // 64):
            pltpu.sync_copy(zbuf, acc.at[pl.ds(s * RPT + kk * 64, 64)])
        plsc.subcore_barrier()

        def issue_idx(j, p):
            pltpu.async_copy(src_hbm.at[wid].at[j], isb[p], si[p])
            pltpu.async_copy(dst_hbm.at[wid].at[j], idb[p], si[p])

        def wait_idx(p):
            pltpu.make_async_copy(src_hbm.at[wid].at[0], isb[p], si[p]).wait()
            pltpu.make_async_copy(dst_hbm.at[wid].at[0], idb[p], si[p]).wait()

        def issue_gather(p):
            pltpu.async_copy(h_hbm.at[isb[p]], rows[p], sg[p])

        def wait_gather(p):
            pltpu.make_async_copy(h_hbm.at[isb[p]], rows[p], sg[p]).wait()

        issue_idx(0, 0)
        wait_idx(0)
        issue_gather(0)
        issue_idx(1, 1)

        def step(j, p, next_gather, idx_prefetch):
            q = 1 - p
            wait_gather(p)
            if next_gather:
                wait_idx(q)
                issue_gather(q)
            pltpu.sync_copy(rows[p], acc.at[idb[p]], add=True)
            if idx_prefetch:
                issue_idx(j + 2, p)

        @pl.loop(0, NCH // 2 - 1)
        def _(t):
            step(2 * t, 0, True, True)
            step(2 * t + 1, 1, True, True)

        step(NCH - 2, 0, True, False)
        step(NCH - 1, 1, False, False)

        plsc.subcore_barrier()
        for kk in range(RPT // 64):
            r0 = s * RPT + kk * 64
            pltpu.sync_copy(acc.at[pl.ds(r0, 64)], out_hbm.at[c].at[pl.ds(r0, 64)])

    return k(h, src3, dst3)


def _sc_decode(ta, tb, i03, i13):
    @functools.partial(
        pl.kernel,
        out_type=jax.ShapeDtypeStruct((EPAD // 8, 128), jnp.float32),
        mesh=_mesh,
        scratch_types=[
            pltpu.VMEM((CH, 16), jnp.float32),
            pltpu.VMEM((CH, 16), jnp.float32),
            pltpu.VMEM((CH, 16), jnp.float32),
            pltpu.VMEM((CH, 16), jnp.float32),
            pltpu.VMEM((CH // 8, 128), jnp.float32),
            pltpu.VMEM((CH // 8, 128), jnp.float32),
            pltpu.VMEM((NCH, CH), jnp.int32),
            pltpu.VMEM((NCH, CH), jnp.int32),
            pltpu.SemaphoreType.DMA,
            pltpu.SemaphoreType.DMA,
            pltpu.SemaphoreType.DMA,
            pltpu.SemaphoreType.DMA,
            pltpu.SemaphoreType.DMA,
            pltpu.SemaphoreType.DMA,
        ],
        compiler_params=_sc_params,
    )
    def k(ta_hbm, tb_hbm, i0_hbm, i1_hbm, out_hbm, va0, vb0, va1, vb1,
          ob0, ob1, idxa, idxb, sa0, sb0, sa1, sb1, sw0, sw1):
        c = lax.axis_index("c")
        s = lax.axis_index("s")
        wid = c * NS + s
        base = wid * EPT

        pltpu.sync_copy(i0_hbm.at[wid], idxa)
        pltpu.sync_copy(i1_hbm.at[wid], idxb)

        va = (va0, va1)
        vb = (vb0, vb1)
        ob = (ob0, ob1)
        sa = (sa0, sa1)
        sb = (sb0, sb1)
        sw = (sw0, sw1)

        def out_slice(j):
            return out_hbm.at[pl.ds((base + j * CH) // 8, CH // 8)]

        def step(j, p, issue_next, guard):
            q = 1 - p
            pltpu.make_async_copy(ta_hbm.at[idxa.at[j]], va[p], sa[p]).wait()
            pltpu.make_async_copy(tb_hbm.at[idxb.at[j]], vb[p], sb[p]).wait()
            if issue_next:
                pltpu.async_copy(ta_hbm.at[idxa.at[j + 1]], va[q], sa[q])
                pltpu.async_copy(tb_hbm.at[idxb.at[j + 1]], vb[q], sb[q])

            def _wait_prev_write():
                pltpu.make_async_copy(ob[p], out_slice(j), sw[p]).wait()

            if guard is None:
                _wait_prev_write()
            else:
                pl.when(guard)(_wait_prev_write)

            @pl.loop(0, CH // 8)
            def _(r):
                for u in range(8):
                    ob[p][r, pl.ds(16 * u, 16)] = va[p][8 * r + u, :] + vb[p][8 * r + u, :]

            pltpu.async_copy(ob[p], out_slice(j), sw[p])

        pltpu.async_copy(ta_hbm.at[idxa.at[0]], va0, sa0)
        pltpu.async_copy(tb_hbm.at[idxb.at[0]], vb0, sb0)

        @pl.loop(0, NCH // 2 - 1)
        def _(t):
            step(2 * t, 0, True, t > 0)
            step(2 * t + 1, 1, True, t > 0)

        step(NCH - 2, 0, True, None)
        step(NCH - 1, 1, False, None)
        pltpu.make_async_copy(ob0, out_slice(NCH - 2), sw0).wait()
        pltpu.make_async_copy(ob1, out_slice(NCH - 1), sw1).wait()

    return k(ta, tb, i03, i13)


_R = 1024


def _tc_prep(degpair, xp, W1):
    def body(dp_ref, x_ref, w_ref, h_ref, rb_ref):
        deg = dp_ref[0][:, 0:1] + dp_ref[1][:, 0:1]
        r = jnp.where(deg > 0, lax.rsqrt(jnp.maximum(deg, 1.0)), 0.0)
        rb = jnp.broadcast_to(r, (_R, D))
        h = jnp.dot(x_ref[...], w_ref[...], preferred_element_type=jnp.float32)
        h_ref[...] = h * rb
        rb_ref[...] = rb

    return pl.pallas_call(
        body,
        grid=(NP // _R,),
        in_specs=[
            pl.BlockSpec((NC, _R, 16), lambda i: (0, i, 0)),
            pl.BlockSpec((_R, D), lambda i: (i, 0)),
            pl.BlockSpec((D, D), lambda i: (0, 0)),
        ],
        out_specs=[
            pl.BlockSpec((_R, D), lambda i: (i, 0)),
            pl.BlockSpec((_R, D), lambda i: (i, 0)),
        ],
        out_shape=[
            jax.ShapeDtypeStruct((NP, D), jnp.float32),
            jax.ShapeDtypeStruct((NP, D), jnp.float32),
        ],
    )(degpair, xp, W1)


def _tc_mid(zpair, rb, W2):
    def body(zp_ref, rb_ref, w_ref, h_ref):
        rbv = rb_ref[...]
        z = jnp.maximum((zp_ref[0] + zp_ref[1]) * rbv, 0.0)
        h_ref[...] = jnp.dot(z, w_ref[...], preferred_element_type=jnp.float32) * rbv

    return pl.pallas_call(
        body,
        grid=(NP // _R,),
        in_specs=[
            pl.BlockSpec((NC, _R, D), lambda i: (0, i, 0)),
            pl.BlockSpec((_R, D), lambda i: (i, 0)),
            pl.BlockSpec((D, D), lambda i: (0, 0)),
        ],
        out_specs=pl.BlockSpec((_R, D), lambda i: (i, 0)),
        out_shape=jax.ShapeDtypeStruct((NP, D), jnp.float32),
    )(zpair, rb, W2)


def _tc_final(zpair, rb, WA, WB):
    def body(zp_ref, rb_ref, wa_ref, wb_ref, ta_ref, tb_ref):
        z = (zp_ref[0] + zp_ref[1]) * rb_ref[...]
        ta_ref[...] = jnp.dot(z, wa_ref[...], preferred_element_type=jnp.float32)
        tb_ref[...] = jnp.dot(z, wb_ref[...], preferred_element_type=jnp.float32)

    return pl.pallas_call(
        body,
        grid=(NP // _R,),
        in_specs=[
            pl.BlockSpec((NC, _R, D), lambda i: (0, i, 0)),
            pl.BlockSpec((_R, D), lambda i: (i, 0)),
            pl.BlockSpec((D, 16), lambda i: (0, 0)),
            pl.BlockSpec((D, 16), lambda i: (0, 0)),
        ],
        out_specs=[
            pl.BlockSpec((_R, 16), lambda i: (i, 0)),
            pl.BlockSpec((_R, 16), lambda i: (i, 0)),
        ],
        out_shape=[
            jax.ShapeDtypeStruct((NP, 16), jnp.float32),
            jax.ShapeDtypeStruct((NP, 16), jnp.float32),
        ],
    )(zpair, rb, WA, WB)


def _pad_edges(idx, fill):
    return jnp.concatenate([idx, fill]).reshape(NW, NCH, CH)


def kernel(x, edge_index, pos_edge_index, neg_edge_index, W1, W2, Wlin):
    pad_nodes = N + (jnp.arange(EPAD - E, dtype=jnp.int32) % (NP - N))
    src3 = _pad_edges(edge_index[0], pad_nodes)
    dst3 = _pad_edges(edge_index[1], pad_nodes)
    pad_rows = jnp.arange(EPAD - EP2, dtype=jnp.int32) % N
    i03 = jnp.concatenate(
        [pos_edge_index[0], neg_edge_index[0], pad_rows]).reshape(NW, NCH, CH)
    i13 = jnp.concatenate(
        [pos_edge_index[1], neg_edge_index[1], pad_rows]).reshape(NW, NCH, CH)
    xp = jnp.pad(x, ((0, NP - N), (0, 0)))
    WA = jnp.pad(Wlin[:, :D].T, ((0, 0), (0, 14)))
    WB = jnp.pad(Wlin[:, D:].T, ((0, 0), (0, 14)))

    degpair = _sc_deg(dst3)
    h1, rb = _tc_prep(degpair, xp, W1)
    z1p = _sc_conv(h1, src3, dst3)
    h2 = _tc_mid(z1p, rb, W2)
    z2p = _sc_conv(h2, src3, dst3)
    ta, tb = _tc_final(z2p, rb, WA, WB)
    outp = _sc_decode(ta, tb, i03, i13)
    return outp.reshape(EPAD // 8, 8, 16)[:, :, :2].reshape(EPAD, 2)[:EP2]
